# baseline (speedup 1.0000x reference)
import jax
import jax.numpy as jnp
from jax import lax
from jax.experimental import pallas as pl
from jax.experimental.pallas import tpu as pltpu
from jax.experimental.pallas import tpu_sc as plsc

N = 10000
E = 160000
DIN = 256
H = 8
DH = 32
HID = H * DH
G = 64

NT = 16
EPT = E // NT
C = 50
NCH = EPT // C
NQ = NCH // 8
QB0 = 12
SZ = 624
TAIL = N - NT * SZ
BM = 1000
NBLK = N // BM


def _tc1_body(x_ref, w_ref, aa_ref, ab_ref, hlo_ref, hhi_ref, oa_ref, ob_ref):
    h = jnp.dot(x_ref[...], w_ref[...], preferred_element_type=jnp.float32)
    hlo_ref[...] = h[:, :128]
    hhi_ref[...] = h[:, 128:]
    oa_ref[...] = jnp.dot(h, aa_ref[...], preferred_element_type=jnp.float32)
    ob_ref[...] = jnp.dot(h, ab_ref[...], preferred_element_type=jnp.float32)


def _tc_layer1(x, w, aa, ab, interpret=False):
    return pl.pallas_call(
        _tc1_body,
        grid=(NBLK,),
        in_specs=[
            pl.BlockSpec((BM, DIN), lambda i: (i, 0)),
            pl.BlockSpec((DIN, HID), lambda i: (0, 0)),
            pl.BlockSpec((HID, 128), lambda i: (0, 0)),
            pl.BlockSpec((HID, 128), lambda i: (0, 0)),
        ],
        out_specs=[
            pl.BlockSpec((BM, 128), lambda i: (i, 0)),
            pl.BlockSpec((BM, 128), lambda i: (i, 0)),
            pl.BlockSpec((BM, 128), lambda i: (i, 0)),
            pl.BlockSpec((BM, 128), lambda i: (i, 0)),
        ],
        out_shape=[
            jax.ShapeDtypeStruct((N, 128), jnp.float32),
            jax.ShapeDtypeStruct((N, 128), jnp.float32),
            jax.ShapeDtypeStruct((N, 128), jnp.float32),
            jax.ShapeDtypeStruct((N, 128), jnp.float32),
        ],
        interpret=interpret,
    )(x, w, aa, ab)


def _normalize(acc0, acc1, den0, den1, sel, b, slope):
    dsum = den0 + den1
    halves = []
    for c, a in enumerate((acc0, acc1)):
        d4 = dsum[:, c * 4:(c + 1) * 4]
        rexp = jnp.dot(1.0 / (d4 + 1e-16), sel,
                       preferred_element_type=jnp.float32)
        xh = a * rexp + b[:, c * 128:(c + 1) * 128]
        if slope is not None:
            xh = jnp.where(xh >= 0, xh, slope * xh)
        halves.append(xh)
    return jnp.concatenate(halves, axis=1)


def _tc2_body(acc0_ref, acc1_ref, den0_ref, den1_ref, b_ref, sel_ref, w_ref,
              aa_ref, ab_ref, hlo_ref, hhi_ref, oa_ref, ob_ref):
    x = _normalize(acc0_ref[0], acc1_ref[0], den0_ref[0], den1_ref[0],
                   sel_ref[...], b_ref[...], 0.01)
    h = jnp.dot(x, w_ref[...], preferred_element_type=jnp.float32)
    hlo_ref[...] = h[:, :128]
    hhi_ref[...] = h[:, 128:]
    oa_ref[...] = jnp.dot(h, aa_ref[...], preferred_element_type=jnp.float32)
    ob_ref[...] = jnp.dot(h, ab_ref[...], preferred_element_type=jnp.float32)


def _tc_layer2(acc, den, b, sel, w, aa, ab, interpret=False):
    return pl.pallas_call(
        _tc2_body,
        grid=(NBLK,),
        in_specs=[
            pl.BlockSpec((1, BM, 128), lambda i: (0, i, 0)),
            pl.BlockSpec((1, BM, 128), lambda i: (1, i, 0)),
            pl.BlockSpec((1, BM, 128), lambda i: (0, i, 0)),
            pl.BlockSpec((1, BM, 128), lambda i: (1, i, 0)),
            pl.BlockSpec((1, HID), lambda i: (0, 0)),
            pl.BlockSpec((4, 128), lambda i: (0, 0)),
            pl.BlockSpec((HID, HID), lambda i: (0, 0)),
            pl.BlockSpec((HID, 128), lambda i: (0, 0)),
            pl.BlockSpec((HID, 128), lambda i: (0, 0)),
        ],
        out_specs=[
            pl.BlockSpec((BM, 128), lambda i: (i, 0)),
            pl.BlockSpec((BM, 128), lambda i: (i, 0)),
            pl.BlockSpec((BM, 128), lambda i: (i, 0)),
            pl.BlockSpec((BM, 128), lambda i: (i, 0)),
        ],
        out_shape=[
            jax.ShapeDtypeStruct((N, 128), jnp.float32),
            jax.ShapeDtypeStruct((N, 128), jnp.float32),
            jax.ShapeDtypeStruct((N, 128), jnp.float32),
            jax.ShapeDtypeStruct((N, 128), jnp.float32),
        ],
        interpret=interpret,
    )(acc, acc, den, den, b, sel, w, aa, ab)


def _pool_body(acc0_ref, acc1_ref, den0_ref, den1_ref, b_ref, sel_ref,
               batch_ref, out_ref, sums, cnt):
    i = pl.program_id(0)

    @pl.when(i == 0)
    def _():
        sums[...] = jnp.zeros_like(sums)
        cnt[...] = jnp.zeros_like(cnt)

    x = _normalize(acc0_ref[0], acc1_ref[0], den0_ref[0], den1_ref[0],
                   sel_ref[...], b_ref[...], None)
    oh = (lax.broadcasted_iota(jnp.int32, (G, BM), 0)
          == batch_ref[0]).astype(jnp.float32)
    sums[...] += jnp.dot(oh, x, preferred_element_type=jnp.float32)
    cnt[...] += jnp.sum(oh, axis=1, keepdims=True)

    @pl.when(i == NBLK - 1)
    def _():
        c1 = jnp.clip(cnt[:, 0:1], 1.0, None)
        out_ref[...] = sums[...] / c1


def _tc_pool(acc, den, b, sel, batch_r, interpret=False):
    return pl.pallas_call(
        _pool_body,
        grid=(NBLK,),
        in_specs=[
            pl.BlockSpec((1, BM, 128), lambda i: (0, i, 0)),
            pl.BlockSpec((1, BM, 128), lambda i: (1, i, 0)),
            pl.BlockSpec((1, BM, 128), lambda i: (0, i, 0)),
            pl.BlockSpec((1, BM, 128), lambda i: (1, i, 0)),
            pl.BlockSpec((1, HID), lambda i: (0, 0)),
            pl.BlockSpec((4, 128), lambda i: (0, 0)),
            pl.BlockSpec((1, 1, BM), lambda i: (i, 0, 0)),
        ],
        out_specs=pl.BlockSpec((G, HID), lambda i: (0, 0)),
        out_shape=jax.ShapeDtypeStruct((G, HID), jnp.float32),
        scratch_shapes=[
            pltpu.VMEM((G, HID), jnp.float32),
            pltpu.VMEM((G, 1), jnp.float32),
        ],
        interpret=interpret,
    )(acc, acc, den, den, b, sel, batch_r)


def _zero_stripe(buf, shared, tid):
    zv = jnp.zeros((16,), jnp.float32)

    def zrow(k, _):
        buf[k // 8, pl.ds((k % 8) * 16, 16)] = zv
        return 0
    lax.fori_loop(0, C * 8, zrow, 0)
    r0 = tid * SZ
    for p in range(SZ // 48):
        pltpu.sync_copy(buf.at[pl.ds(0, 48)],
                        shared.at[pl.ds(r0 + p * 48, 48)])

    @pl.when(tid == NT - 1)
    def _():
        pltpu.sync_copy(buf.at[pl.ds(0, TAIL)],
                        shared.at[pl.ds(NT * SZ, TAIL)])


def _copy_stripe_out(shared, out_ref, cid, tid):
    r0 = tid * SZ
    for p in range(SZ // 208):
        pltpu.sync_copy(shared.at[pl.ds(r0 + p * 208, 208)],
                        out_ref.at[cid, pl.ds(r0 + p * 208, 208)])

    @pl.when(tid == NT - 1)
    def _():
        pltpu.sync_copy(shared.at[pl.ds(NT * SZ, TAIL)],
                        out_ref.at[cid, pl.ds(NT * SZ, TAIL)])


def _sc_den_body(ei_ref, aa_ref, ab_ref,
                 den_out, w_out,
                 den_s, sidx8, didx8, asb, adb, wtile, sem1, sem2):
    cid = lax.axis_index("c")
    tid = lax.axis_index("s")

    _zero_stripe(adb, den_s, tid)
    plsc.subcore_barrier()

    q0 = cid * QB0
    nq = jnp.where(cid == 0, QB0, NQ - QB0)

    def block(qq, _):
        q = q0 + qq
        pltpu.sync_copy(ei_ref.at[0, tid, q], sidx8)
        pltpu.sync_copy(ei_ref.at[1, tid, q], didx8)
        pltpu.async_copy(aa_ref.at[sidx8.at[0]], asb.at[pl.ds(0, C)], sem1)
        pltpu.async_copy(ab_ref.at[didx8.at[0]], adb.at[pl.ds(0, C)], sem2)

        def chunk(jj, _):
            par = jj & 1
            nxt = (jj + 1) & 1

            @pl.when(jj < 7)
            def _():
                pltpu.async_copy(aa_ref.at[sidx8.at[jj + 1]],
                                 asb.at[pl.ds(nxt * 56, C)], sem1)
                pltpu.async_copy(ab_ref.at[didx8.at[jj + 1]],
                                 adb.at[pl.ds(nxt * 56, C)], sem2)

            pltpu.make_async_copy(aa_ref.at[sidx8.at[jj]],
                                  asb.at[pl.ds(par * 56, C)], sem1).wait()
            pltpu.make_async_copy(ab_ref.at[didx8.at[jj]],
                                  adb.at[pl.ds(par * 56, C)], sem2).wait()

            def edge(e, _):
                r = par * 56 + e
                full = asb[r, pl.ds(0, 16)] + adb[r, pl.ds(0, 16)]
                lk = jnp.where(full >= 0.0, full, 0.2 * full)
                w = jnp.exp(lk)
                asb[r, pl.ds(0, 16)] = w
                wtile[e // 8, pl.ds((e % 8) * 16, 16)] = w
                return 0
            lax.fori_loop(0, C, edge, 0, unroll=5)
            pltpu.sync_copy(asb.at[pl.ds(par * 56, C)],
                            den_s.at[didx8.at[jj]], add=True)
            pltpu.sync_copy(wtile, w_out.at[tid, q * 8 + jj])
            return 0
        lax.fori_loop(0, 8, chunk, 0)
        return 0
    lax.fori_loop(0, nq, block, 0)
    plsc.subcore_barrier()
    _copy_stripe_out(den_s, den_out, cid, tid)


def _sc_den(ei_r, aa, ab, interpret=False):
    mesh = plsc.VectorSubcoreMesh(core_axis_name="c", subcore_axis_name="s",
                                  num_cores=2, num_subcores=NT)
    f = pl.kernel(
        _sc_den_body,
        out_type=[
            jax.ShapeDtypeStruct((2, N, 128), jnp.float32),
            jax.ShapeDtypeStruct((NT, NCH, 8, 128), jnp.float32),
        ],
        mesh=mesh,
        scratch_types=[
            pltpu.VMEM_SHARED((N, 128), jnp.float32),
            pltpu.VMEM((8, C), jnp.int32),
            pltpu.VMEM((8, C), jnp.int32),
            pltpu.VMEM((112, 128), jnp.float32),
            pltpu.VMEM((112, 128), jnp.float32),
            pltpu.VMEM((8, 128), jnp.float32),
            pltpu.SemaphoreType.DMA,
            pltpu.SemaphoreType.DMA,
        ],
        compiler_params=pltpu.CompilerParams(needs_layout_passes=False),
        interpret=interpret,
    )
    return f(ei_r, aa, ab)


def _sc_msg_body(ei_ref, hlo_ref, hhi_ref, w_ref,
                 acc_out,
                 acc_s, sidx8, didx8, msgb, wtb, gsem, wsem, ssem):
    cid = lax.axis_index("c")
    tid = lax.axis_index("s")

    _zero_stripe(msgb, acc_s, tid)
    plsc.subcore_barrier()

    def edge_pass(base, h_ref):
        def go():
            def drain_one_scatter():
                pltpu.make_async_copy(msgb.at[pl.ds(0, C)],
                                      acc_s.at[didx8.at[0]], ssem).wait()

            def block(q, _):
                pltpu.sync_copy(ei_ref.at[0, tid, q], sidx8)
                pltpu.sync_copy(ei_ref.at[1, tid, q], didx8)

                @pl.when(q > 0)
                def _():
                    drain_one_scatter()
                pltpu.async_copy(h_ref.at[sidx8.at[0]],
                                 msgb.at[pl.ds(0, C)], gsem)
                pltpu.async_copy(w_ref.at[tid, q * 8],
                                 wtb.at[pl.ds(0, 8)], wsem)

                def chunk(jj, _):
                    par = jj & 1
                    nxt = (jj + 1) & 1

                    @pl.when((jj < 7) & (q * 8 + jj >= 1))
                    def _():
                        drain_one_scatter()

                    @pl.when(jj < 7)
                    def _():
                        pltpu.async_copy(h_ref.at[sidx8.at[jj + 1]],
                                         msgb.at[pl.ds(nxt * 56, C)], gsem)
                        pltpu.async_copy(w_ref.at[tid, q * 8 + jj + 1],
                                         wtb.at[pl.ds(nxt * 8, 8)], wsem)

                    pltpu.make_async_copy(h_ref.at[sidx8.at[jj]],
                                          msgb.at[pl.ds(par * 56, C)],
                                          gsem).wait()
                    pltpu.make_async_copy(w_ref.at[tid, q * 8 + jj],
                                          wtb.at[pl.ds(par * 8, 8)],
                                          wsem).wait()

                    def edge(e, _):
                        w = wtb[par * 8 + e // 8, pl.ds((e % 8) * 16, 16)]
                        r = par * 56 + e
                        for hh in range(4):
                            wl = w[base + hh]
                            for half in range(2):
                                sl = pl.ds(hh * 32 + half * 16, 16)
                                msgb[r, sl] = msgb[r, sl] * wl
                        return 0
                    lax.fori_loop(0, C, edge, 0, unroll=5)
                    pltpu.async_copy(msgb.at[pl.ds(par * 56, C)],
                                     acc_s.at[didx8.at[jj]], ssem, add=True)
                    return 0
                lax.fori_loop(0, 8, chunk, 0)
                return 0
            lax.fori_loop(0, NQ, block, 0)
            drain_one_scatter()
            drain_one_scatter()
        return go

    pl.when(cid == 0)(edge_pass(0, hlo_ref))
    pl.when(cid == 1)(edge_pass(4, hhi_ref))
    plsc.subcore_barrier()
    _copy_stripe_out(acc_s, acc_out, cid, tid)


def _sc_msg(ei_r, hlo, hhi, w_all, interpret=False):
    mesh = plsc.VectorSubcoreMesh(core_axis_name="c", subcore_axis_name="s",
                                  num_cores=2, num_subcores=NT)
    f = pl.kernel(
        _sc_msg_body,
        out_type=[
            jax.ShapeDtypeStruct((2, N, 128), jnp.float32),
        ],
        mesh=mesh,
        scratch_types=[
            pltpu.VMEM_SHARED((N, 128), jnp.float32),
            pltpu.VMEM((8, C), jnp.int32),
            pltpu.VMEM((8, C), jnp.int32),
            pltpu.VMEM((112, 128), jnp.float32),
            pltpu.VMEM((16, 128), jnp.float32),
            pltpu.SemaphoreType.DMA,
            pltpu.SemaphoreType.DMA,
            pltpu.SemaphoreType.DMA,
        ],
        compiler_params=pltpu.CompilerParams(needs_layout_passes=False),
        interpret=interpret,
    )
    return f(ei_r, hlo, hhi, w_all)[0]


def _attn_mats(a_src, a_dst):
    eye = jnp.eye(H, dtype=jnp.float32)
    msrc = (eye[:, None, :] * a_src[:, :, None]).reshape(HID, H)
    mdst = (eye[:, None, :] * a_dst[:, :, None]).reshape(HID, H)
    z = jnp.zeros((HID, 128 - 2 * H), jnp.float32)
    return (jnp.concatenate([msrc, mdst, z], axis=1),
            jnp.concatenate([mdst, msrc, z], axis=1))


def kernel(edge_index, node_feat, batch, W1, a_src1, a_dst1, b1,
           W2, a_src2, a_dst2, b2):
    ei_r = edge_index.reshape(2, NT, NQ, 8, C)
    aa1, ab1 = _attn_mats(a_src1, a_dst1)
    aa2, ab2 = _attn_mats(a_src2, a_dst2)
    sel = jnp.kron(jnp.eye(4, dtype=jnp.float32),
                   jnp.ones((1, DH), jnp.float32))
    hlo, hhi, oa, ob = _tc_layer1(node_feat, W1, aa1, ab1)
    den1, w1 = _sc_den(ei_r, oa, ob)
    acc1 = _sc_msg(ei_r, hlo, hhi, w1)
    hlo2, hhi2, oa2, ob2 = _tc_layer2(acc1, den1, b1.reshape(1, HID), sel,
                                      W2, aa2, ab2)
    den2, w2 = _sc_den(ei_r, oa2, ob2)
    acc2 = _sc_msg(ei_r, hlo2, hhi2, w2)
    return _tc_pool(acc2, den2, b2.reshape(1, HID), sel,
                    batch.reshape(NBLK, 1, BM))


# speedup vs baseline: 1.0208x; 1.0208x over previous
import jax
import jax.numpy as jnp
from jax import lax
from jax.experimental import pallas as pl
from jax.experimental.pallas import tpu as pltpu
from jax.experimental.pallas import tpu_sc as plsc

N = 10000
E = 160000
DIN = 256
H = 8
DH = 32
HID = H * DH
G = 64

NT = 16
EPT = E // NT
C = 50
NCH = EPT // C
NQ = NCH // 8
QB0 = 12
SZ = 624
TAIL = N - NT * SZ
BM = 1000
NBLK = N // BM


def _tc1_body(x_ref, w_ref, aa_ref, ab_ref, hlo_ref, hhi_ref, oa_ref, ob_ref):
    h = jnp.dot(x_ref[...], w_ref[...], preferred_element_type=jnp.float32)
    hlo_ref[...] = h[:, :128]
    hhi_ref[...] = h[:, 128:]
    oa_ref[...] = jnp.dot(h, aa_ref[...], preferred_element_type=jnp.float32)
    ob_ref[...] = jnp.dot(h, ab_ref[...], preferred_element_type=jnp.float32)


def _tc_layer1(x, w, aa, ab, interpret=False):
    return pl.pallas_call(
        _tc1_body,
        grid=(NBLK,),
        in_specs=[
            pl.BlockSpec((BM, DIN), lambda i: (i, 0)),
            pl.BlockSpec((DIN, HID), lambda i: (0, 0)),
            pl.BlockSpec((HID, 128), lambda i: (0, 0)),
            pl.BlockSpec((HID, 128), lambda i: (0, 0)),
        ],
        out_specs=[
            pl.BlockSpec((BM, 128), lambda i: (i, 0)),
            pl.BlockSpec((BM, 128), lambda i: (i, 0)),
            pl.BlockSpec((BM, 128), lambda i: (i, 0)),
            pl.BlockSpec((BM, 128), lambda i: (i, 0)),
        ],
        out_shape=[
            jax.ShapeDtypeStruct((N, 128), jnp.float32),
            jax.ShapeDtypeStruct((N, 128), jnp.float32),
            jax.ShapeDtypeStruct((N, 128), jnp.float32),
            jax.ShapeDtypeStruct((N, 128), jnp.float32),
        ],
        interpret=interpret,
    )(x, w, aa, ab)


def _normalize(acc0, acc1, den0, den1, sel, b, slope):
    dsum = den0 + den1
    halves = []
    for c, a in enumerate((acc0, acc1)):
        d4 = dsum[:, c * 4:(c + 1) * 4]
        rexp = jnp.dot(1.0 / (d4 + 1e-16), sel,
                       preferred_element_type=jnp.float32)
        xh = a * rexp + b[:, c * 128:(c + 1) * 128]
        if slope is not None:
            xh = jnp.where(xh >= 0, xh, slope * xh)
        halves.append(xh)
    return jnp.concatenate(halves, axis=1)


def _tc2_body(acc0_ref, acc1_ref, den0_ref, den1_ref, b_ref, sel_ref, w_ref,
              aa_ref, ab_ref, hlo_ref, hhi_ref, oa_ref, ob_ref):
    x = _normalize(acc0_ref[0], acc1_ref[0], den0_ref[0], den1_ref[0],
                   sel_ref[...], b_ref[...], 0.01)
    h = jnp.dot(x, w_ref[...], preferred_element_type=jnp.float32)
    hlo_ref[...] = h[:, :128]
    hhi_ref[...] = h[:, 128:]
    oa_ref[...] = jnp.dot(h, aa_ref[...], preferred_element_type=jnp.float32)
    ob_ref[...] = jnp.dot(h, ab_ref[...], preferred_element_type=jnp.float32)


def _tc_layer2(acc, den, b, sel, w, aa, ab, interpret=False):
    return pl.pallas_call(
        _tc2_body,
        grid=(NBLK,),
        in_specs=[
            pl.BlockSpec((1, BM, 128), lambda i: (0, i, 0)),
            pl.BlockSpec((1, BM, 128), lambda i: (1, i, 0)),
            pl.BlockSpec((1, BM, 128), lambda i: (0, i, 0)),
            pl.BlockSpec((1, BM, 128), lambda i: (1, i, 0)),
            pl.BlockSpec((1, HID), lambda i: (0, 0)),
            pl.BlockSpec((4, 128), lambda i: (0, 0)),
            pl.BlockSpec((HID, HID), lambda i: (0, 0)),
            pl.BlockSpec((HID, 128), lambda i: (0, 0)),
            pl.BlockSpec((HID, 128), lambda i: (0, 0)),
        ],
        out_specs=[
            pl.BlockSpec((BM, 128), lambda i: (i, 0)),
            pl.BlockSpec((BM, 128), lambda i: (i, 0)),
            pl.BlockSpec((BM, 128), lambda i: (i, 0)),
            pl.BlockSpec((BM, 128), lambda i: (i, 0)),
        ],
        out_shape=[
            jax.ShapeDtypeStruct((N, 128), jnp.float32),
            jax.ShapeDtypeStruct((N, 128), jnp.float32),
            jax.ShapeDtypeStruct((N, 128), jnp.float32),
            jax.ShapeDtypeStruct((N, 128), jnp.float32),
        ],
        interpret=interpret,
    )(acc, acc, den, den, b, sel, w, aa, ab)


def _pool_body(acc0_ref, acc1_ref, den0_ref, den1_ref, b_ref, sel_ref,
               batch_ref, out_ref, sums, cnt):
    i = pl.program_id(0)

    @pl.when(i == 0)
    def _():
        sums[...] = jnp.zeros_like(sums)
        cnt[...] = jnp.zeros_like(cnt)

    x = _normalize(acc0_ref[0], acc1_ref[0], den0_ref[0], den1_ref[0],
                   sel_ref[...], b_ref[...], None)
    oh = (lax.broadcasted_iota(jnp.int32, (G, BM), 0)
          == batch_ref[0]).astype(jnp.float32)
    sums[...] += jnp.dot(oh, x, preferred_element_type=jnp.float32)
    cnt[...] += jnp.sum(oh, axis=1, keepdims=True)

    @pl.when(i == NBLK - 1)
    def _():
        c1 = jnp.clip(cnt[:, 0:1], 1.0, None)
        out_ref[...] = sums[...] / c1


def _tc_pool(acc, den, b, sel, batch_r, interpret=False):
    return pl.pallas_call(
        _pool_body,
        grid=(NBLK,),
        in_specs=[
            pl.BlockSpec((1, BM, 128), lambda i: (0, i, 0)),
            pl.BlockSpec((1, BM, 128), lambda i: (1, i, 0)),
            pl.BlockSpec((1, BM, 128), lambda i: (0, i, 0)),
            pl.BlockSpec((1, BM, 128), lambda i: (1, i, 0)),
            pl.BlockSpec((1, HID), lambda i: (0, 0)),
            pl.BlockSpec((4, 128), lambda i: (0, 0)),
            pl.BlockSpec((1, 1, BM), lambda i: (i, 0, 0)),
        ],
        out_specs=pl.BlockSpec((G, HID), lambda i: (0, 0)),
        out_shape=jax.ShapeDtypeStruct((G, HID), jnp.float32),
        scratch_shapes=[
            pltpu.VMEM((G, HID), jnp.float32),
            pltpu.VMEM((G, 1), jnp.float32),
        ],
        interpret=interpret,
    )(acc, acc, den, den, b, sel, batch_r)


def _zero_stripe(buf, shared, tid):
    zv = jnp.zeros((16,), jnp.float32)

    def zrow(k, _):
        buf[k // 8, pl.ds((k % 8) * 16, 16)] = zv
        return 0
    lax.fori_loop(0, C * 8, zrow, 0)
    r0 = tid * SZ
    for p in range(SZ // 48):
        pltpu.sync_copy(buf.at[pl.ds(0, 48)],
                        shared.at[pl.ds(r0 + p * 48, 48)])

    @pl.when(tid == NT - 1)
    def _():
        pltpu.sync_copy(buf.at[pl.ds(0, TAIL)],
                        shared.at[pl.ds(NT * SZ, TAIL)])


def _copy_stripe_out(shared, out_ref, cid, tid):
    r0 = tid * SZ
    for p in range(SZ // 208):
        pltpu.sync_copy(shared.at[pl.ds(r0 + p * 208, 208)],
                        out_ref.at[cid, pl.ds(r0 + p * 208, 208)])

    @pl.when(tid == NT - 1)
    def _():
        pltpu.sync_copy(shared.at[pl.ds(NT * SZ, TAIL)],
                        out_ref.at[cid, pl.ds(NT * SZ, TAIL)])


def _sc_den_body(ei_ref, aa_ref, ab_ref,
                 den_out, w_out,
                 den_s, sidx8, didx8, asb, adb, wtile, sem1, sem2, ssem):
    cid = lax.axis_index("c")
    tid = lax.axis_index("s")

    _zero_stripe(adb, den_s, tid)
    plsc.subcore_barrier()

    q0 = cid * QB0
    nq = jnp.where(cid == 0, QB0, NQ - QB0)

    def drain_one_scatter():
        pltpu.make_async_copy(asb.at[pl.ds(0, C)],
                              den_s.at[didx8.at[0]], ssem).wait()

    def block(qq, _):
        q = q0 + qq
        pltpu.sync_copy(ei_ref.at[0, tid, q], sidx8)
        pltpu.sync_copy(ei_ref.at[1, tid, q], didx8)

        @pl.when(qq > 0)
        def _():
            drain_one_scatter()
        pltpu.async_copy(aa_ref.at[sidx8.at[0]], asb.at[pl.ds(0, C)], sem1)
        pltpu.async_copy(ab_ref.at[didx8.at[0]], adb.at[pl.ds(0, C)], sem2)

        def chunk(jj, _):
            par = jj & 1
            nxt = (jj + 1) & 1

            @pl.when((jj < 7) & (qq * 8 + jj >= 1))
            def _():
                drain_one_scatter()

            @pl.when(jj < 7)
            def _():
                pltpu.async_copy(aa_ref.at[sidx8.at[jj + 1]],
                                 asb.at[pl.ds(nxt * 56, C)], sem1)
                pltpu.async_copy(ab_ref.at[didx8.at[jj + 1]],
                                 adb.at[pl.ds(nxt * 56, C)], sem2)

            pltpu.make_async_copy(aa_ref.at[sidx8.at[jj]],
                                  asb.at[pl.ds(par * 56, C)], sem1).wait()
            pltpu.make_async_copy(ab_ref.at[didx8.at[jj]],
                                  adb.at[pl.ds(par * 56, C)], sem2).wait()

            def edge(e, _):
                r = par * 56 + e
                full = asb[r, pl.ds(0, 16)] + adb[r, pl.ds(0, 16)]
                lk = jnp.where(full >= 0.0, full, 0.2 * full)
                w = jnp.exp(lk)
                asb[r, pl.ds(0, 16)] = w
                wtile[e // 8, pl.ds((e % 8) * 16, 16)] = w
                return 0
            lax.fori_loop(0, C, edge, 0, unroll=5)
            pltpu.async_copy(asb.at[pl.ds(par * 56, C)],
                             den_s.at[didx8.at[jj]], ssem, add=True)
            pltpu.sync_copy(wtile, w_out.at[tid, q * 8 + jj])
            return 0
        lax.fori_loop(0, 8, chunk, 0)
        return 0
    lax.fori_loop(0, nq, block, 0)
    drain_one_scatter()
    drain_one_scatter()
    plsc.subcore_barrier()
    _copy_stripe_out(den_s, den_out, cid, tid)


def _sc_den(ei_r, aa, ab, interpret=False):
    mesh = plsc.VectorSubcoreMesh(core_axis_name="c", subcore_axis_name="s",
                                  num_cores=2, num_subcores=NT)
    f = pl.kernel(
        _sc_den_body,
        out_type=[
            jax.ShapeDtypeStruct((2, N, 128), jnp.float32),
            jax.ShapeDtypeStruct((NT, NCH, 8, 128), jnp.float32),
        ],
        mesh=mesh,
        scratch_types=[
            pltpu.VMEM_SHARED((N, 128), jnp.float32),
            pltpu.VMEM((8, C), jnp.int32),
            pltpu.VMEM((8, C), jnp.int32),
            pltpu.VMEM((112, 128), jnp.float32),
            pltpu.VMEM((112, 128), jnp.float32),
            pltpu.VMEM((8, 128), jnp.float32),
            pltpu.SemaphoreType.DMA,
            pltpu.SemaphoreType.DMA,
            pltpu.SemaphoreType.DMA,
        ],
        compiler_params=pltpu.CompilerParams(needs_layout_passes=False),
        interpret=interpret,
    )
    return f(ei_r, aa, ab)


def _sc_msg_body(ei_ref, hlo_ref, hhi_ref, w_ref,
                 acc_out,
                 acc_s, sidx8, didx8, msgb, wtb, gsem, wsem, ssem):
    cid = lax.axis_index("c")
    tid = lax.axis_index("s")

    _zero_stripe(msgb, acc_s, tid)
    plsc.subcore_barrier()

    def edge_pass(base, h_ref):
        def go():
            def drain_one_scatter():
                pltpu.make_async_copy(msgb.at[pl.ds(0, C)],
                                      acc_s.at[didx8.at[0]], ssem).wait()

            def block(q, _):
                pltpu.sync_copy(ei_ref.at[0, tid, q], sidx8)
                pltpu.sync_copy(ei_ref.at[1, tid, q], didx8)

                @pl.when(q > 0)
                def _():
                    drain_one_scatter()
                pltpu.async_copy(h_ref.at[sidx8.at[0]],
                                 msgb.at[pl.ds(0, C)], gsem)
                pltpu.async_copy(w_ref.at[tid, q * 8],
                                 wtb.at[pl.ds(0, 8)], wsem)

                def chunk(jj, _):
                    par = jj & 1
                    nxt = (jj + 1) & 1

                    @pl.when((jj < 7) & (q * 8 + jj >= 1))
                    def _():
                        drain_one_scatter()

                    @pl.when(jj < 7)
                    def _():
                        pltpu.async_copy(h_ref.at[sidx8.at[jj + 1]],
                                         msgb.at[pl.ds(nxt * 56, C)], gsem)
                        pltpu.async_copy(w_ref.at[tid, q * 8 + jj + 1],
                                         wtb.at[pl.ds(nxt * 8, 8)], wsem)

                    pltpu.make_async_copy(h_ref.at[sidx8.at[jj]],
                                          msgb.at[pl.ds(par * 56, C)],
                                          gsem).wait()
                    pltpu.make_async_copy(w_ref.at[tid, q * 8 + jj],
                                          wtb.at[pl.ds(par * 8, 8)],
                                          wsem).wait()

                    def edge(e, _):
                        w = wtb[par * 8 + e // 8, pl.ds((e % 8) * 16, 16)]
                        r = par * 56 + e
                        for hh in range(4):
                            wl = w[base + hh]
                            for half in range(2):
                                sl = pl.ds(hh * 32 + half * 16, 16)
                                msgb[r, sl] = msgb[r, sl] * wl
                        return 0
                    lax.fori_loop(0, C, edge, 0, unroll=5)
                    pltpu.async_copy(msgb.at[pl.ds(par * 56, C)],
                                     acc_s.at[didx8.at[jj]], ssem, add=True)
                    return 0
                lax.fori_loop(0, 8, chunk, 0)
                return 0
            lax.fori_loop(0, NQ, block, 0)
            drain_one_scatter()
            drain_one_scatter()
        return go

    pl.when(cid == 0)(edge_pass(0, hlo_ref))
    pl.when(cid == 1)(edge_pass(4, hhi_ref))
    plsc.subcore_barrier()
    _copy_stripe_out(acc_s, acc_out, cid, tid)


def _sc_msg(ei_r, hlo, hhi, w_all, interpret=False):
    mesh = plsc.VectorSubcoreMesh(core_axis_name="c", subcore_axis_name="s",
                                  num_cores=2, num_subcores=NT)
    f = pl.kernel(
        _sc_msg_body,
        out_type=[
            jax.ShapeDtypeStruct((2, N, 128), jnp.float32),
        ],
        mesh=mesh,
        scratch_types=[
            pltpu.VMEM_SHARED((N, 128), jnp.float32),
            pltpu.VMEM((8, C), jnp.int32),
            pltpu.VMEM((8, C), jnp.int32),
            pltpu.VMEM((112, 128), jnp.float32),
            pltpu.VMEM((16, 128), jnp.float32),
            pltpu.SemaphoreType.DMA,
            pltpu.SemaphoreType.DMA,
            pltpu.SemaphoreType.DMA,
        ],
        compiler_params=pltpu.CompilerParams(needs_layout_passes=False),
        interpret=interpret,
    )
    return f(ei_r, hlo, hhi, w_all)[0]


def _attn_mats(a_src, a_dst):
    eye = jnp.eye(H, dtype=jnp.float32)
    msrc = (eye[:, None, :] * a_src[:, :, None]).reshape(HID, H)
    mdst = (eye[:, None, :] * a_dst[:, :, None]).reshape(HID, H)
    z = jnp.zeros((HID, 128 - 2 * H), jnp.float32)
    return (jnp.concatenate([msrc, mdst, z], axis=1),
            jnp.concatenate([mdst, msrc, z], axis=1))


def kernel(edge_index, node_feat, batch, W1, a_src1, a_dst1, b1,
           W2, a_src2, a_dst2, b2):
    ei_r = edge_index.reshape(2, NT, NQ, 8, C)
    aa1, ab1 = _attn_mats(a_src1, a_dst1)
    aa2, ab2 = _attn_mats(a_src2, a_dst2)
    sel = jnp.kron(jnp.eye(4, dtype=jnp.float32),
                   jnp.ones((1, DH), jnp.float32))
    hlo, hhi, oa, ob = _tc_layer1(node_feat, W1, aa1, ab1)
    den1, w1 = _sc_den(ei_r, oa, ob)
    acc1 = _sc_msg(ei_r, hlo, hhi, w1)
    hlo2, hhi2, oa2, ob2 = _tc_layer2(acc1, den1, b1.reshape(1, HID), sel,
                                      W2, aa2, ab2)
    den2, w2 = _sc_den(ei_r, oa2, ob2)
    acc2 = _sc_msg(ei_r, hlo2, hhi2, w2)
    return _tc_pool(acc2, den2, b2.reshape(1, HID), sel,
                    batch.reshape(NBLK, 1, BM))
